# baseline (speedup 1.0000x reference)
import functools
import math

import jax
import jax.numpy as jnp
from jax import lax
from jax.experimental import pallas as pl
from jax.experimental.pallas import tpu as pltpu
from jax.experimental.pallas import tpu_sc as plsc

D_MODEL = 64
SCALE = math.sqrt(D_MODEL)
LANES = 16
G = 512
NBUF = 2


def _emb_body(n_per_w, num_cores, x_hbm, lut_hbm, out_hbm, *refs):
    idx = refs[0:NBUF]
    rows = refs[NBUF:2 * NBUF]
    si = refs[2 * NBUF:3 * NBUF]
    sg = refs[3 * NBUF:4 * NBUF]
    so = refs[4 * NBUF:5 * NBUF]

    wid = lax.axis_index("s") * num_cores + lax.axis_index("c")
    base = wid * n_per_w
    ng = n_per_w // G
    n_outer = ng // NBUF

    for b in range(NBUF):
        pltpu.async_copy(x_hbm.at[pl.ds(base + b * G, G)], idx[b], si[b])

    def outer(gg, _):
        for b in range(NBUF):
            @pl.when(gg > 0)
            def _wait_out():
                pltpu.make_async_copy(rows[b], out_hbm.at[pl.ds(base, G)],
                                      so[b]).wait()
            pltpu.make_async_copy(x_hbm.at[pl.ds(base, G)], idx[b],
                                  si[b]).wait()
            pltpu.async_copy(lut_hbm.at[idx[b]], rows[b], sg[b])
        for b in range(NBUF):
            g = gg * NBUF + b
            pltpu.make_async_copy(lut_hbm.at[pl.ds(0, G)], rows[b],
                                  sg[b]).wait()

            @pl.when(gg < n_outer - 1)
            def _refill_idx():
                pltpu.async_copy(x_hbm.at[pl.ds(base + (g + NBUF) * G, G)],
                                 idx[b], si[b])

            def mul(i, _):
                for r in range(8):
                    row = i * 8 + r
                    for q in range(D_MODEL // LANES):
                        sl = pl.ds(q * LANES, LANES)
                        rows[b][row, sl] = rows[b][row, sl] * SCALE
                return 0

            lax.fori_loop(0, G // 8, mul, 0)
            pltpu.async_copy(rows[b], out_hbm.at[pl.ds(base + g * G, G)],
                             so[b])
        return 0

    lax.fori_loop(0, n_outer, outer, 0)
    for b in range(NBUF):
        pltpu.make_async_copy(rows[b], out_hbm.at[pl.ds(base, G)],
                              so[b]).wait()


def kernel(x, lut):
    b, t = x.shape
    n = b * t
    x1d = x.reshape(n).astype(jnp.int32)

    info = plsc.get_sparse_core_info()
    num_workers = info.num_cores * info.num_subcores
    n_per_w = n // num_workers
    assert n_per_w * num_workers == n
    assert n_per_w % (G * NBUF) == 0

    mesh = plsc.VectorSubcoreMesh(core_axis_name="c", subcore_axis_name="s")
    body = functools.partial(_emb_body, n_per_w, info.num_cores)

    out = pl.kernel(
        body,
        mesh=mesh,
        compiler_params=pltpu.CompilerParams(use_tc_tiling_on_sc=False),
        out_type=jax.ShapeDtypeStruct((n, D_MODEL), jnp.float32),
        scratch_types=(
            [pltpu.VMEM((G,), jnp.int32) for _ in range(NBUF)]
            + [pltpu.VMEM((G, D_MODEL), jnp.float32) for _ in range(NBUF)]
            + [pltpu.SemaphoreType.DMA for _ in range(3 * NBUF)]
        ),
    )(x1d, lut)
    return out.reshape(b, t, D_MODEL)
